# baseline (speedup 1.0000x reference)
import jax
import jax.numpy as jnp
from jax.experimental import pallas as pl
from jax.experimental.pallas import tpu as pltpu

E = 16
N_GROUP = 8
TOPK_GROUP = 4
H = 1024
I = 1024
T = 2048
CH = 256


def _router_body(x_ref, gw_ref, b_ref, w_ref):
    x = x_ref[...]
    logits = jax.lax.dot_general(
        x, gw_ref[...], (((1,), (1,)), ((), ())),
        preferred_element_type=jnp.float32)
    s = jax.nn.sigmoid(logits)
    sfc = s + b_ref[...]
    lane = jax.lax.broadcasted_iota(jnp.int32, (T, E), 1)
    left = pltpu.roll(sfc, E - 1, 1)
    right = pltpu.roll(sfc, 1, 1)
    partner = jnp.where(lane % 2 == 0, left, right)
    ggs = sfc + partner
    glane = lane // 2
    cnt = jnp.zeros((T, E), jnp.int32)
    for j in range(N_GROUP):
        b = ggs[:, 2 * j:2 * j + 1]
        beats = (b > ggs) | ((b == ggs) & (j < glane))
        cnt = cnt + beats.astype(jnp.int32)
    mask = (cnt < TOPK_GROUP).astype(jnp.float32)
    wsel = s * mask
    norm = jnp.sum(wsel, axis=1, keepdims=True)
    w_ref[...] = wsel / norm


def _experts_body(xb_ref, w_ref, w1_ref, w3_ref, w2_ref, o_ref, g_scr):
    e = pl.program_id(0)
    wall = w_ref[...]
    lane = jax.lax.broadcasted_iota(jnp.int32, (T, E), 1)
    tokw = jnp.sum(jnp.where(lane == e, wall, 0.0), axis=1, keepdims=True)
    xb = xb_ref[...]
    w2b = w2_ref[0].astype(jnp.bfloat16)
    tokwb = tokw.astype(jnp.bfloat16)
    for i in range(I // CH):
        sl = slice(i * CH, (i + 1) * CH)
        w13 = jnp.concatenate(
            [w1_ref[0, sl, :], w3_ref[0, sl, :]], axis=0).astype(jnp.bfloat16)
        h13 = jax.lax.dot_general(
            xb, w13, (((1,), (1,)), ((), ())),
            preferred_element_type=jnp.float32)
        h1 = h13[:, :CH]
        h3 = (h13[:, CH:]).astype(jnp.bfloat16)
        s1 = (h1 * jax.nn.sigmoid(h1)).astype(jnp.bfloat16)
        g_scr[:, sl] = s1 * h3 * tokwb
    g = g_scr[...]
    for n in range(H // CH):
        nsl = slice(n * CH, (n + 1) * CH)
        yc = jax.lax.dot_general(
            g, w2b[nsl, :], (((1,), (1,)), ((), ())),
            preferred_element_type=jnp.float32)

        @pl.when(e == 0)
        def _():
            o_ref[:, nsl] = yc

        @pl.when(e != 0)
        def _():
            o_ref[:, nsl] = o_ref[:, nsl] + yc


@jax.jit
def kernel(hidden_states, gate_w, w1, w3, w2, bias):
    bias2d = bias.reshape(1, E)
    routing_w = pl.pallas_call(
        _router_body,
        in_specs=[
            pl.BlockSpec((T, H), lambda: (0, 0)),
            pl.BlockSpec((E, H), lambda: (0, 0)),
            pl.BlockSpec((1, E), lambda: (0, 0)),
        ],
        out_specs=pl.BlockSpec((T, E), lambda: (0, 0)),
        out_shape=jax.ShapeDtypeStruct((T, E), jnp.float32),
    )(hidden_states, gate_w, bias2d)

    xb = hidden_states.astype(jnp.bfloat16)
    out = pl.pallas_call(
        _experts_body,
        grid=(E,),
        in_specs=[
            pl.BlockSpec((T, H), lambda e: (0, 0)),
            pl.BlockSpec((T, E), lambda e: (0, 0)),
            pl.BlockSpec((1, I, H), lambda e: (e, 0, 0)),
            pl.BlockSpec((1, I, H), lambda e: (e, 0, 0)),
            pl.BlockSpec((1, H, I), lambda e: (e, 0, 0)),
        ],
        out_specs=pl.BlockSpec((T, H), lambda e: (0, 0)),
        out_shape=jax.ShapeDtypeStruct((T, H), jnp.float32),
        scratch_shapes=[
            pltpu.VMEM((T, I), jnp.bfloat16),
        ],
        compiler_params=pltpu.CompilerParams(
            dimension_semantics=("arbitrary",),
        ),
    )(xb, routing_w, w1, w3, w2)
    return out


# speedup vs baseline: 1.3688x; 1.3688x over previous
import jax
import jax.numpy as jnp
from jax.experimental import pallas as pl
from jax.experimental.pallas import tpu as pltpu

E = 16
N_GROUP = 8
TOPK_GROUP = 4
H = 1024
I = 1024
T = 2048
CH = 256


def _router_body(x_ref, gw_ref, b_ref, w_ref):
    x = x_ref[...]
    logits = jax.lax.dot_general(
        x, gw_ref[...], (((1,), (1,)), ((), ())),
        preferred_element_type=jnp.float32)
    s = jax.nn.sigmoid(logits)
    sfc = s + b_ref[...]
    lane = jax.lax.broadcasted_iota(jnp.int32, (T, E), 1)
    left = pltpu.roll(sfc, E - 1, 1)
    right = pltpu.roll(sfc, 1, 1)
    partner = jnp.where(lane % 2 == 0, left, right)
    ggs = sfc + partner
    glane = lane // 2
    cnt = jnp.zeros((T, E), jnp.int32)
    for j in range(N_GROUP):
        b = ggs[:, 2 * j:2 * j + 1]
        beats = (b > ggs) | ((b == ggs) & (j < glane))
        cnt = cnt + beats.astype(jnp.int32)
    mask = (cnt < TOPK_GROUP).astype(jnp.float32)
    wsel = s * mask
    norm = jnp.sum(wsel, axis=1, keepdims=True)
    w_ref[...] = wsel / norm


def _experts_body(xb_ref, w_ref, w1_ref, w3_ref, w2_ref, o_ref, g_scr):
    e = pl.program_id(0)
    wall = w_ref[...]
    lane = jax.lax.broadcasted_iota(jnp.int32, (T, E), 1)
    tokw = jnp.sum(jnp.where(lane == e, wall, 0.0), axis=1, keepdims=True)
    @pl.when(e == 0)
    def _():
        o_ref[...] = jnp.zeros((T, H), jnp.float32)

    xb = xb_ref[...]
    w2b = w2_ref[0].astype(jnp.bfloat16)
    tokwb = tokw.astype(jnp.bfloat16)
    for i in range(I // CH):
        sl = slice(i * CH, (i + 1) * CH)
        w13 = jnp.concatenate(
            [w1_ref[0, sl, :], w3_ref[0, sl, :]], axis=0).astype(jnp.bfloat16)
        h13 = jax.lax.dot_general(
            xb, w13, (((1,), (1,)), ((), ())),
            preferred_element_type=jnp.float32)
        h1 = h13[:, :CH]
        h3 = (h13[:, CH:]).astype(jnp.bfloat16)
        s1 = (h1 * jax.nn.sigmoid(h1)).astype(jnp.bfloat16)
        g_scr[:, sl] = s1 * h3 * tokwb
    y = jax.lax.dot_general(
        g_scr[...], w2b, (((1,), (1,)), ((), ())),
        preferred_element_type=jnp.float32)
    o_ref[...] = o_ref[...] + y


@jax.jit
def kernel(hidden_states, gate_w, w1, w3, w2, bias):
    bias2d = bias.reshape(1, E)
    routing_w = pl.pallas_call(
        _router_body,
        in_specs=[
            pl.BlockSpec((T, H), lambda: (0, 0)),
            pl.BlockSpec((E, H), lambda: (0, 0)),
            pl.BlockSpec((1, E), lambda: (0, 0)),
        ],
        out_specs=pl.BlockSpec((T, E), lambda: (0, 0)),
        out_shape=jax.ShapeDtypeStruct((T, E), jnp.float32),
    )(hidden_states, gate_w, bias2d)

    xb = hidden_states.astype(jnp.bfloat16)
    out = pl.pallas_call(
        _experts_body,
        grid=(E,),
        in_specs=[
            pl.BlockSpec((T, H), lambda e: (0, 0)),
            pl.BlockSpec((T, E), lambda e: (0, 0)),
            pl.BlockSpec((1, I, H), lambda e: (e, 0, 0)),
            pl.BlockSpec((1, I, H), lambda e: (e, 0, 0)),
            pl.BlockSpec((1, H, I), lambda e: (e, 0, 0)),
        ],
        out_specs=pl.BlockSpec((T, H), lambda e: (0, 0)),
        out_shape=jax.ShapeDtypeStruct((T, H), jnp.float32),
        scratch_shapes=[
            pltpu.VMEM((T, I), jnp.bfloat16),
        ],
        compiler_params=pltpu.CompilerParams(
            dimension_semantics=("arbitrary",),
        ),
    )(xb, routing_w, w1, w3, w2)
    return out


# speedup vs baseline: 1.4018x; 1.0241x over previous
import jax
import jax.numpy as jnp
from jax.experimental import pallas as pl
from jax.experimental.pallas import tpu as pltpu

E = 16
N_GROUP = 8
TOPK_GROUP = 4
H = 1024
I = 1024
T = 2048
CH = 256


def _router_body(x_ref, gw_ref, b_ref, w_ref, xb_ref):
    x = x_ref[...]
    xb_ref[...] = x.astype(jnp.bfloat16)
    logits = jax.lax.dot_general(
        x, gw_ref[...], (((1,), (1,)), ((), ())),
        preferred_element_type=jnp.float32)
    s = jax.nn.sigmoid(logits)
    sfc = s + b_ref[...]
    lane = jax.lax.broadcasted_iota(jnp.int32, (T, E), 1)
    left = pltpu.roll(sfc, E - 1, 1)
    right = pltpu.roll(sfc, 1, 1)
    partner = jnp.where(lane % 2 == 0, left, right)
    ggs = sfc + partner
    glane = lane // 2
    cnt = jnp.zeros((T, E), jnp.int32)
    for j in range(N_GROUP):
        b = ggs[:, 2 * j:2 * j + 1]
        beats = (b > ggs) | ((b == ggs) & (j < glane))
        cnt = cnt + beats.astype(jnp.int32)
    mask = (cnt < TOPK_GROUP).astype(jnp.float32)
    wsel = s * mask
    norm = jnp.sum(wsel, axis=1, keepdims=True)
    w_ref[...] = wsel / norm


def _experts_body(xb_ref, w_ref, w1_ref, w3_ref, w2_ref, o_ref, g_scr):
    e = pl.program_id(0)
    wall = w_ref[...]
    lane = jax.lax.broadcasted_iota(jnp.int32, (T, E), 1)
    tokw = jnp.sum(jnp.where(lane == e, wall, 0.0), axis=1, keepdims=True)
    @pl.when(e == 0)
    def _():
        o_ref[...] = jnp.zeros((T, H), jnp.float32)

    xb = xb_ref[...]
    w2b = w2_ref[0].astype(jnp.bfloat16)
    tokwb = tokw.astype(jnp.bfloat16)
    for i in range(I // CH):
        sl = slice(i * CH, (i + 1) * CH)
        w13 = jnp.concatenate(
            [w1_ref[0, sl, :], w3_ref[0, sl, :]], axis=0).astype(jnp.bfloat16)
        h13 = jax.lax.dot_general(
            xb, w13, (((1,), (1,)), ((), ())),
            preferred_element_type=jnp.float32)
        h1 = h13[:, :CH]
        h3 = (h13[:, CH:]).astype(jnp.bfloat16)
        s1 = (h1 * jax.nn.sigmoid(h1)).astype(jnp.bfloat16)
        g_scr[:, sl] = s1 * h3 * tokwb
    y = jax.lax.dot_general(
        g_scr[...], w2b, (((1,), (1,)), ((), ())),
        preferred_element_type=jnp.float32)
    o_ref[...] = o_ref[...] + y


@jax.jit
def kernel(hidden_states, gate_w, w1, w3, w2, bias):
    bias2d = bias.reshape(1, E)
    routing_w = pl.pallas_call(
        _router_body,
        in_specs=[
            pl.BlockSpec((T, H), lambda: (0, 0)),
            pl.BlockSpec((E, H), lambda: (0, 0)),
            pl.BlockSpec((1, E), lambda: (0, 0)),
        ],
        out_specs=[
            pl.BlockSpec((T, E), lambda: (0, 0)),
            pl.BlockSpec((T, H), lambda: (0, 0)),
        ],
        out_shape=[
            jax.ShapeDtypeStruct((T, E), jnp.float32),
            jax.ShapeDtypeStruct((T, H), jnp.bfloat16),
        ],
    )(hidden_states, gate_w, bias2d)
    routing_w, xb = routing_w
    out = pl.pallas_call(
        _experts_body,
        grid=(E,),
        in_specs=[
            pl.BlockSpec((T, H), lambda e: (0, 0)),
            pl.BlockSpec((T, E), lambda e: (0, 0)),
            pl.BlockSpec((1, I, H), lambda e: (e, 0, 0)),
            pl.BlockSpec((1, I, H), lambda e: (e, 0, 0)),
            pl.BlockSpec((1, H, I), lambda e: (e, 0, 0)),
        ],
        out_specs=pl.BlockSpec((T, H), lambda e: (0, 0)),
        out_shape=jax.ShapeDtypeStruct((T, H), jnp.float32),
        scratch_shapes=[
            pltpu.VMEM((T, I), jnp.bfloat16),
        ],
        compiler_params=pltpu.CompilerParams(
            dimension_semantics=("arbitrary",),
        ),
    )(xb, routing_w, w1, w3, w2)
    return out
